# baseline (speedup 1.0000x reference)
import functools

import jax
import jax.numpy as jnp
from jax import lax
from jax.experimental import pallas as pl
from jax.experimental.pallas import tpu as pltpu
from jax.experimental.pallas import tpu_sc as plsc

_RB = 512
_KCH = 256
_CHP = 128


def _ceil_to(a, m):
    return (a + m - 1) // m * m


def _lane_shuffle(v, perm):
    dnums = lax.GatherDimensionNumbers(
        offset_dims=(), collapsed_slice_dims=(0,), start_index_map=(0,))
    return lax.gather(v, perm[:, None], dnums, (1,),
                      mode=lax.GatherScatterMode.PROMISE_IN_BOUNDS)


def _tc_first(xp, w0, deg32r, n_real):
    npad, din = xp.shape
    hid = w0.shape[1]
    hh = hid // 2
    nb = npad // _RB

    def body(x_ref, w_ref, *rest):
        dg = rest[:32]
        g_ref, dv_ref = rest[32], rest[33]
        i = pl.program_id(0)
        rows = lax.broadcasted_iota(jnp.int32, (_RB, 1), 0) + i * _RB
        deg = dg[0][:, 0:1] + 1.0
        for k in range(1, 32):
            deg = deg + dg[k][:, 0:1]
        dinv = jnp.where(rows < n_real, lax.rsqrt(deg), 0.0)
        g_ref[...] = jnp.dot(x_ref[...], w_ref[...],
                             preferred_element_type=jnp.float32) * dinv
        dv_ref[...] = jnp.broadcast_to(dinv, (_RB, 8))

    deg_specs = [pl.BlockSpec((_RB, 8), (lambda i, h, k=k: (k * nb + i, 0)))
                 for k in range(32)]
    return pl.pallas_call(
        body,
        grid=(nb, 2),
        in_specs=[pl.BlockSpec((_RB, din), lambda i, h: (i, 0)),
                  pl.BlockSpec((din, hh), lambda i, h: (0, h))] + deg_specs,
        out_specs=[pl.BlockSpec((_RB, hh), lambda i, h: (h * nb + i, 0)),
                   pl.BlockSpec((_RB, 8), lambda i, h: (i, 0))],
        out_shape=[jax.ShapeDtypeStruct((2 * npad, hh), jnp.float32),
                   jax.ShapeDtypeStruct((npad, 8), jnp.float32)],
    )(xp, w0, *([deg32r] * 32))


def _tc_mid(acc2, g2, dinv8, brow, w):
    npad2, hh = g2.shape
    npad = npad2 // 2
    hid = 2 * hh
    nb = npad // _RB

    def body(aa, ab, ga, gb, dv, b_r, w_r, go):
        dinv = dv[:, 0:1]
        acc = jnp.concatenate([aa[...] + ga[...], ab[...] + gb[...]], axis=1)
        h = jnp.maximum(acc * dinv + b_r[...], 0.0)
        go[...] = jnp.dot(h, w_r[...],
                          preferred_element_type=jnp.float32) * dinv

    return pl.pallas_call(
        body,
        grid=(nb, 2),
        in_specs=[pl.BlockSpec((_RB, hh), lambda i, h: (i, 0)),
                  pl.BlockSpec((_RB, hh), lambda i, h: (nb + i, 0)),
                  pl.BlockSpec((_RB, hh), lambda i, h: (i, 0)),
                  pl.BlockSpec((_RB, hh), lambda i, h: (nb + i, 0)),
                  pl.BlockSpec((_RB, 8), lambda i, h: (i, 0)),
                  pl.BlockSpec((1, hid), lambda i, h: (0, 0)),
                  pl.BlockSpec((hid, hh), lambda i, h: (0, h))],
        out_specs=pl.BlockSpec((_RB, hh), lambda i, h: (h * nb + i, 0)),
        out_shape=jax.ShapeDtypeStruct((2 * npad, hh), jnp.float32),
    )(acc2, acc2, g2, g2, dinv8, brow, w)


def _tc_fin(acc2, g2, dinv8, brow, pw1a, pw1b, pb1row):
    npad2, hh = g2.shape
    npad = npad2 // 2
    hid = 2 * hh
    nb = npad // _RB

    def body(aa, ab, ga, gb, dv, b_r, wa_r, wb_r, pb_r, ao, bo):
        dinv = dv[:, 0:1]
        acc = jnp.concatenate([aa[...] + ga[...], ab[...] + gb[...]], axis=1)
        h = jnp.maximum(acc * dinv + b_r[...], 0.0)
        ao[...] = jnp.dot(h, wa_r[...],
                          preferred_element_type=jnp.float32) + pb_r[...]
        bo[...] = jnp.dot(h, wb_r[...], preferred_element_type=jnp.float32)

    return pl.pallas_call(
        body,
        grid=(nb,),
        in_specs=[pl.BlockSpec((_RB, hh), lambda i: (i, 0)),
                  pl.BlockSpec((_RB, hh), lambda i: (nb + i, 0)),
                  pl.BlockSpec((_RB, hh), lambda i: (i, 0)),
                  pl.BlockSpec((_RB, hh), lambda i: (nb + i, 0)),
                  pl.BlockSpec((_RB, 8), lambda i: (i, 0)),
                  pl.BlockSpec((1, hid), lambda i: (0, 0)),
                  pl.BlockSpec((hid, hid), lambda i: (0, 0)),
                  pl.BlockSpec((hid, hid), lambda i: (0, 0)),
                  pl.BlockSpec((1, hid), lambda i: (0, 0))],
        out_specs=[pl.BlockSpec((_RB, hid), lambda i: (i, 0)),
                   pl.BlockSpec((_RB, hid), lambda i: (i, 0))],
        out_shape=[jax.ShapeDtypeStruct((npad, hid), jnp.float32),
                   jax.ShapeDtypeStruct((npad, hid), jnp.float32)],
    )(acc2, acc2, g2, g2, dinv8, brow, pw1a, pw1b, pb1row)


def _sc_deg(dstp, zacc8, npad):
    epad = dstp.shape[0]
    ept = epad // 32
    nit = ept // _KCH
    nrow8 = (npad + 2) * 8
    rpt8 = (npad // 16) * 8
    mesh = plsc.VectorSubcoreMesh(core_axis_name="c", subcore_axis_name="s")

    @functools.partial(
        pl.kernel, mesh=mesh,
        out_type=jax.ShapeDtypeStruct((32 * npad * 8,), jnp.float32),
        scratch_types=[pltpu.VMEM((_KCH,), jnp.int32),
                       pltpu.VMEM((nrow8,), jnp.float32)],
    )
    def k(dst_h, z_h, deg_h, dstv, accp):
        c = lax.axis_index("c")
        s = lax.axis_index("s")
        w = s * 2 + c
        pltpu.sync_copy(z_h, accp)
        lanes = lax.broadcasted_iota(jnp.int32, (16,), 0)
        v_lo = jnp.where(lanes < 8, 1.0, 0.0).astype(jnp.float32)
        v_hi = jnp.where(lanes < 8, 0.0, 1.0).astype(jnp.float32)

        def chunk(it, carry):
            base = w * ept + it * _KCH
            pltpu.sync_copy(dst_h.at[pl.ds(base, _KCH)], dstv)

            def grp(i, carry2):
                dv16 = dstv[pl.ds(i * 16, 16)]
                lo16 = (dv16 + 1) * 8
                hi16 = dv16 * 8
                for kk in range(0, 16, 2):
                    plsc.addupdate(accp.at[pl.ds(lo16[kk], 16)], v_lo)
                    plsc.addupdate(accp.at[pl.ds(hi16[kk + 1], 16)], v_hi)
                return carry2

            lax.fori_loop(0, _KCH // 16, grp, 0)
            return carry

        lax.fori_loop(0, nit, chunk, 0)
        pltpu.sync_copy(accp.at[pl.ds(8, npad * 8)],
                        deg_h.at[pl.ds(w * npad * 8, npad * 8)])

    return k(dstp, zacc8)


def _sc_scat(g2, srcp, dstp, zacc, npad):
    epad = dstp.shape[0]
    nit = epad // _KCH
    half = npad // 2
    mesh = plsc.VectorSubcoreMesh(core_axis_name="c", subcore_axis_name="s")

    @functools.partial(
        pl.kernel, mesh=mesh,
        out_type=jax.ShapeDtypeStruct((32 * half * 16,), jnp.float32),
        scratch_types=[pltpu.VMEM((_KCH,), jnp.int32),
                       pltpu.VMEM((_KCH,), jnp.int32),
                       pltpu.VMEM((_KCH // 128, 128), jnp.int32),
                       pltpu.VMEM((_KCH, 128), jnp.float32),
                       pltpu.VMEM(((half + 2) * 16,), jnp.float32),
                       pltpu.SemaphoreType.DMA],
    )
    def k(g_h, src_h, dst_h, z_h, acc_h, srcv, dstv, srcadj, rows, accp,
          sem):
        c = lax.axis_index("c")
        s = lax.axis_index("s")
        f2 = s % 8
        nh = s // 8
        cbase = c * npad
        nodebase = nh * half
        foff = f2 * 16
        pltpu.sync_copy(z_h, accp)

        def chunk(it, carry):
            base = it * _KCH
            pltpu.sync_copy(src_h.at[pl.ds(base, _KCH)], srcv)
            pltpu.sync_copy(dst_h.at[pl.ds(base, _KCH)], dstv)
            for j in range(_KCH // 16):
                v = srcv[pl.ds(j * 16, 16)] + cbase
                srcadj[j // 8, pl.ds((j % 8) * 16, 16)] = v
            cps = [pltpu.async_copy(g_h.at[srcadj.at[q]],
                                    rows.at[pl.ds(q * 128, 128)], sem)
                   for q in range(_KCH // 128)]
            for cp in cps:
                cp.wait()

            def grp(i, carry2):
                i0 = i * 16
                dv16 = dstv[pl.ds(i0, 16)]
                dloc = dv16 - nodebase + 1
                own = jnp.logical_and(dloc >= 1, dloc <= half)
                idx16 = jnp.where(own, dloc, 0) * 16
                for kk in range(16):
                    plsc.addupdate(accp.at[pl.ds(idx16[kk], 16)],
                                   rows[i0 + kk, pl.ds(foff, 16)])
                return carry2

            lax.fori_loop(0, _KCH // 16, grp, 0)
            return carry

        lax.fori_loop(0, nit, chunk, 0)
        q = c * 16 + s
        pltpu.sync_copy(accp.at[pl.ds(16, half * 16)],
                        acc_h.at[pl.ds(q * half * 16, half * 16)])

    return k(g2, srcp, dstp, zacc)


def _sc_pair(aarr, barr, psp, pdp, pw2flat):
    npad, hid = aarr.shape
    ppad = psp.shape[0]
    ppt = ppad // 32
    nit = ppt // _CHP
    nj = hid // 16
    mesh = plsc.VectorSubcoreMesh(core_axis_name="c", subcore_axis_name="s")

    @functools.partial(
        pl.kernel, mesh=mesh,
        out_type=jax.ShapeDtypeStruct((ppad,), jnp.float32),
        scratch_types=[pltpu.VMEM((_CHP,), jnp.int32),
                       pltpu.VMEM((_CHP,), jnp.int32),
                       pltpu.VMEM((_CHP, hid), jnp.float32),
                       pltpu.VMEM((_CHP, hid), jnp.float32),
                       pltpu.VMEM((_CHP,), jnp.float32),
                       pltpu.VMEM((hid,), jnp.float32),
                       pltpu.SemaphoreType.DMA,
                       pltpu.SemaphoreType.DMA],
    )
    def k(a_h, b_h, ps_h, pd_h, pw2_h, out_h,
          psv, pdv, ra, rb, sv, pw2v, semA, semB):
        c = lax.axis_index("c")
        s = lax.axis_index("s")
        w = s * 2 + c
        pltpu.sync_copy(pw2_h, pw2v)

        def chunk(it, carry):
            base = w * ppt + it * _CHP
            pltpu.sync_copy(ps_h.at[pl.ds(base, _CHP)], psv)
            pltpu.sync_copy(pd_h.at[pl.ds(base, _CHP)], pdv)
            cpa = pltpu.async_copy(a_h.at[psv], ra, semA)
            cpb = pltpu.async_copy(b_h.at[pdv], rb, semB)
            cpa.wait()
            cpb.wait()

            lanes = lax.broadcasted_iota(jnp.int32, (16,), 0)
            perms = [jnp.bitwise_xor(lanes, sh) for sh in (8, 4, 2, 1)]

            def group(gi, carry2):
                i0 = gi * 16
                svec = jnp.zeros((16,), jnp.float32)
                for kk in range(16):
                    i = i0 + kk
                    acc = jnp.zeros((16,), jnp.float32)
                    for j in range(nj):
                        va = ra[i, pl.ds(j * 16, 16)]
                        vb = rb[i, pl.ds(j * 16, 16)]
                        acc = acc + jnp.maximum(va + vb, 0.0) \
                            * pw2v[pl.ds(j * 16, 16)]
                    for perm in perms:
                        acc = acc + _lane_shuffle(acc, perm)
                    svec = jnp.where(lanes == kk, acc, svec)
                sv[pl.ds(gi * 16, 16)] = svec
                return carry2

            lax.fori_loop(0, _CHP // 16, group, 0)
            pltpu.sync_copy(sv, out_h.at[pl.ds(base, _CHP)])
            return carry

        lax.fori_loop(0, nit, chunk, 0)

    return k(aarr, barr, psp, pdp, pw2flat)


def kernel(x, edge_index, pairs, W0, b0, W1, b1, W2, b2, PW1, Pb1, PW2, Pb2):
    N, din = x.shape
    hid = W0.shape[1]
    E = edge_index.shape[1]
    P = pairs.shape[0]

    npad = _ceil_to(N + 1, 2 * _RB)
    epad = _ceil_to(E, 32 * _KCH)
    ppad = _ceil_to(P, 32 * _CHP)
    half = npad // 2

    src = edge_index[0].astype(jnp.int32)
    dst = edge_index[1].astype(jnp.int32)
    ndum = npad - N
    if epad > E:
        fill = N + jnp.arange(epad - E, dtype=jnp.int32) % ndum
        src = jnp.concatenate([src, fill])
        dst = jnp.concatenate([dst, fill])
    ps = pairs[:, 0].astype(jnp.int32)
    pd = pairs[:, 1].astype(jnp.int32)
    if ppad > P:
        pfill = N + jnp.arange(ppad - P, dtype=jnp.int32) % ndum
        ps = jnp.concatenate([ps, pfill])
        pd = jnp.concatenate([pd, pfill])

    xp = jnp.pad(x, ((0, npad - N), (0, 0)))
    zacc8 = jnp.zeros(((npad + 2) * 8,), jnp.float32)
    zacc = jnp.zeros(((half + 2) * 16,), jnp.float32)

    def scat(g2_):
        flat = _sc_scat(g2_, src, dst, zacc, npad)
        return (flat.reshape(2, 2, 8, half, 16)
                .transpose(0, 1, 3, 2, 4).reshape(2 * npad, 128))
    b0r = b0.reshape(1, hid)
    b1r = b1.reshape(1, hid)
    b2r = b2.reshape(1, hid)
    pb1r = Pb1.reshape(1, hid)
    pw1a = PW1[:hid]
    pw1b = PW1[hid:]
    pw2flat = PW2[:, 0]

    deg32r = _sc_deg(dst, zacc8, npad).reshape(32 * npad, 8)
    g2, dinv8 = _tc_first(xp, W0, deg32r, N)
    acc2 = scat(g2)
    g2 = _tc_mid(acc2, g2, dinv8, b0r, W1)
    acc2 = scat(g2)
    g2 = _tc_mid(acc2, g2, dinv8, b1r, W2)
    acc2 = scat(g2)
    aarr, barr = _tc_fin(acc2, g2, dinv8, b2r, pw1a, pw1b, pb1r)
    scores = _sc_pair(aarr, barr, ps, pd, pw2flat)
    return scores[:P] + Pb2[0]


# speedup vs baseline: 1.6879x; 1.6879x over previous
import functools

import jax
import numpy as np
import jax.numpy as jnp
from jax import lax
from jax.experimental import pallas as pl
from jax.experimental.pallas import tpu as pltpu
from jax.experimental.pallas import tpu_sc as plsc

_RB = 512
_KCH = 256
_CHP = 128


def _ceil_to(a, m):
    return (a + m - 1) // m * m


def _lane_shuffle(v, perm):
    dnums = lax.GatherDimensionNumbers(
        offset_dims=(), collapsed_slice_dims=(0,), start_index_map=(0,))
    return lax.gather(v, perm[:, None], dnums, (1,),
                      mode=lax.GatherScatterMode.PROMISE_IN_BOUNDS)


def _tc_first(xp, w0, deg32r, n_real):
    npad, din = xp.shape
    hid = w0.shape[1]
    hh = hid // 2
    nb = npad // _RB

    def body(x_ref, w_ref, *rest):
        dg = rest[:32]
        g_ref, dv_ref = rest[32], rest[33]
        i = pl.program_id(0)
        rows = lax.broadcasted_iota(jnp.int32, (_RB, 1), 0) + i * _RB
        deg = dg[0][:, 0:1] + 1.0
        for k in range(1, 32):
            deg = deg + dg[k][:, 0:1]
        dinv = jnp.where(rows < n_real, lax.rsqrt(deg), 0.0)
        g_ref[...] = jnp.dot(x_ref[...], w_ref[...],
                             preferred_element_type=jnp.float32) * dinv
        dv_ref[...] = jnp.broadcast_to(dinv, (_RB, 8))

    deg_specs = [pl.BlockSpec((_RB, 8), (lambda i, h, k=k: (k * nb + i, 0)))
                 for k in range(32)]
    return pl.pallas_call(
        body,
        grid=(nb, 2),
        in_specs=[pl.BlockSpec((_RB, din), lambda i, h: (i, 0)),
                  pl.BlockSpec((din, hh), lambda i, h: (0, h))] + deg_specs,
        out_specs=[pl.BlockSpec((_RB, hh), lambda i, h: (h * nb + i, 0)),
                   pl.BlockSpec((_RB, 8), lambda i, h: (i, 0))],
        out_shape=[jax.ShapeDtypeStruct((2 * npad, hh), jnp.float32),
                   jax.ShapeDtypeStruct((npad, 8), jnp.float32)],
    )(xp, w0, *([deg32r] * 32))


def _tc_mid(acc2, g2, dinv8, brow, w):
    npad2, hh = g2.shape
    npad = npad2 // 2
    hid = 2 * hh
    nb = npad // _RB

    def body(aa, ab, ga, gb, dv, b_r, w_r, go):
        dinv = dv[:, 0:1]
        acc = jnp.concatenate([aa[...] + ga[...], ab[...] + gb[...]], axis=1)
        h = jnp.maximum(acc * dinv + b_r[...], 0.0)
        go[...] = jnp.dot(h, w_r[...],
                          preferred_element_type=jnp.float32) * dinv

    return pl.pallas_call(
        body,
        grid=(nb, 2),
        in_specs=[pl.BlockSpec((_RB, hh), lambda i, h: (i, 0)),
                  pl.BlockSpec((_RB, hh), lambda i, h: (nb + i, 0)),
                  pl.BlockSpec((_RB, hh), lambda i, h: (i, 0)),
                  pl.BlockSpec((_RB, hh), lambda i, h: (nb + i, 0)),
                  pl.BlockSpec((_RB, 8), lambda i, h: (i, 0)),
                  pl.BlockSpec((1, hid), lambda i, h: (0, 0)),
                  pl.BlockSpec((hid, hh), lambda i, h: (0, h))],
        out_specs=pl.BlockSpec((_RB, hh), lambda i, h: (h * nb + i, 0)),
        out_shape=jax.ShapeDtypeStruct((2 * npad, hh), jnp.float32),
    )(acc2, acc2, g2, g2, dinv8, brow, w)


def _tc_fin(acc2, g2, dinv8, brow, pw1a, pw1b, pb1row):
    npad2, hh = g2.shape
    npad = npad2 // 2
    hid = 2 * hh
    nb = npad // _RB

    def body(aa, ab, ga, gb, dv, b_r, wa_r, wb_r, pb_r, ao, bo):
        dinv = dv[:, 0:1]
        acc = jnp.concatenate([aa[...] + ga[...], ab[...] + gb[...]], axis=1)
        h = jnp.maximum(acc * dinv + b_r[...], 0.0)
        ao[...] = jnp.dot(h, wa_r[...],
                          preferred_element_type=jnp.float32) + pb_r[...]
        bo[...] = jnp.dot(h, wb_r[...], preferred_element_type=jnp.float32)

    return pl.pallas_call(
        body,
        grid=(nb,),
        in_specs=[pl.BlockSpec((_RB, hh), lambda i: (i, 0)),
                  pl.BlockSpec((_RB, hh), lambda i: (nb + i, 0)),
                  pl.BlockSpec((_RB, hh), lambda i: (i, 0)),
                  pl.BlockSpec((_RB, hh), lambda i: (nb + i, 0)),
                  pl.BlockSpec((_RB, 8), lambda i: (i, 0)),
                  pl.BlockSpec((1, hid), lambda i: (0, 0)),
                  pl.BlockSpec((hid, hid), lambda i: (0, 0)),
                  pl.BlockSpec((hid, hid), lambda i: (0, 0)),
                  pl.BlockSpec((1, hid), lambda i: (0, 0))],
        out_specs=[pl.BlockSpec((_RB, hid), lambda i: (i, 0)),
                   pl.BlockSpec((_RB, hid), lambda i: (i, 0))],
        out_shape=[jax.ShapeDtypeStruct((npad, hid), jnp.float32),
                   jax.ShapeDtypeStruct((npad, hid), jnp.float32)],
    )(acc2, acc2, g2, g2, dinv8, brow, pw1a, pw1b, pb1row)


def _sc_deg(dstp, zacc8, npad):
    epad = dstp.shape[0]
    ept = epad // 32
    nit = ept // _KCH
    nrow8 = (npad + 2) * 8
    rpt8 = (npad // 16) * 8
    mesh = plsc.VectorSubcoreMesh(core_axis_name="c", subcore_axis_name="s")

    @functools.partial(
        pl.kernel, mesh=mesh,
        out_type=jax.ShapeDtypeStruct((32 * npad * 8,), jnp.float32),
        scratch_types=[pltpu.VMEM((_KCH,), jnp.int32),
                       pltpu.VMEM((nrow8,), jnp.float32)],
    )
    def k(dst_h, z_h, deg_h, dstv, accp):
        c = lax.axis_index("c")
        s = lax.axis_index("s")
        w = s * 2 + c
        pltpu.sync_copy(z_h, accp)
        lanes = lax.broadcasted_iota(jnp.int32, (16,), 0)
        v_lo = jnp.where(lanes < 8, 1.0, 0.0).astype(jnp.float32)
        v_hi = jnp.where(lanes < 8, 0.0, 1.0).astype(jnp.float32)

        def chunk(it, carry):
            base = w * ept + it * _KCH
            pltpu.sync_copy(dst_h.at[pl.ds(base, _KCH)], dstv)

            def grp(i, carry2):
                dv16 = dstv[pl.ds(i * 16, 16)]
                lo16 = (dv16 + 1) * 8
                hi16 = dv16 * 8
                for kk in range(0, 16, 2):
                    plsc.addupdate(accp.at[pl.ds(lo16[kk], 16)], v_lo)
                    plsc.addupdate(accp.at[pl.ds(hi16[kk + 1], 16)], v_hi)
                return carry2

            lax.fori_loop(0, _KCH // 16, grp, 0)
            return carry

        lax.fori_loop(0, nit, chunk, 0)
        pltpu.sync_copy(accp.at[pl.ds(8, npad * 8)],
                        deg_h.at[pl.ds(w * npad * 8, npad * 8)])

    return k(dstp, zacc8)


def _sc_scat(g2, srcp, dstp, zacc, lut, npad):
    epad = dstp.shape[0]
    nit = epad // _KCH
    rpt = npad // 16
    mesh = plsc.VectorSubcoreMesh(core_axis_name="c", subcore_axis_name="s")

    @functools.partial(
        pl.kernel, mesh=mesh,
        out_type=jax.ShapeDtypeStruct((32 * rpt * 128,), jnp.float32),
        scratch_types=[pltpu.VMEM((_KCH,), jnp.int32),
                       pltpu.VMEM((_KCH,), jnp.int32),
                       pltpu.VMEM((192,), jnp.int32),
                       pltpu.VMEM((192,), jnp.int32),
                       pltpu.VMEM((256, 128), jnp.int32),
                       pltpu.VMEM((64, 128), jnp.float32),
                       pltpu.VMEM(((rpt + 1) * 128,), jnp.float32),
                       pltpu.SemaphoreType.DMA],
    )
    def k(g_h, src_h, dst_h, z_h, lut_h, acc_h, srcv, dstv, srcf, dstf,
          lutv, rows, accp, sem):
        c = lax.axis_index("c")
        s = lax.axis_index("s")
        cbase = c * npad
        nodebase = s * rpt
        pltpu.sync_copy(z_h, accp)
        pltpu.sync_copy(lut_h, lutv)
        lanes = lax.broadcasted_iota(jnp.int32, (16,), 0)
        pow2 = jnp.left_shift(1, lanes % 8)
        perms_h = [jnp.bitwise_xor(lanes, sh) for sh in (4, 2, 1)]

        def flush():
            cp = pltpu.async_copy(g_h.at[srcf.at[pl.ds(0, 64)]], rows, sem)
            cp.wait()

            def grp(i, carry2):
                i0 = i * 16
                dl16 = dstf[pl.ds(i0, 16)] * 128
                for kk in range(16):
                    base2 = dl16[kk]
                    for j in range(8):
                        plsc.addupdate(
                            accp.at[pl.ds(base2 + j * 16, 16)],
                            rows[i0 + kk, pl.ds(j * 16, 16)])
                return carry2

            lax.fori_loop(0, 4, grp, 0)

        def chunk(it, cnt):
            base = it * _KCH
            pltpu.sync_copy(src_h.at[pl.ds(base, _KCH)], srcv)
            pltpu.sync_copy(dst_h.at[pl.ds(base, _KCH)], dstv)

            def grp(i, cnt2):
                i0 = i * 16
                dv16 = dstv[pl.ds(i0, 16)]
                sv16 = srcv[pl.ds(i0, 16)] + cbase
                dloc = dv16 - nodebase + 1
                own = jnp.logical_and(dloc >= 1, dloc <= rpt)
                mv = jnp.where(own, pow2, 0)
                for perm in perms_h:
                    mv = mv + _lane_shuffle(mv, perm)
                lr0 = lutv[mv[0], pl.ds(0, 16)]
                lr1 = lutv[mv[8], pl.ds(0, 16)]
                srcf[pl.ds(cnt2, 16)] = _lane_shuffle(sv16, lr0)
                dstf[pl.ds(cnt2, 16)] = _lane_shuffle(dloc, lr0)
                cnt2 = cnt2 + lr0[8]
                perm1 = lr1 + 8
                srcf[pl.ds(cnt2, 16)] = _lane_shuffle(sv16, perm1)
                dstf[pl.ds(cnt2, 16)] = _lane_shuffle(dloc, perm1)
                cnt2 = cnt2 + lr1[8]

                @pl.when(cnt2 >= 64)
                def _():
                    flush()
                    for t in range(2):
                        tail_s = srcf[pl.ds(64 + t * 16, 16)]
                        tail_d = dstf[pl.ds(64 + t * 16, 16)]
                        srcf[pl.ds(t * 16, 16)] = tail_s
                        dstf[pl.ds(t * 16, 16)] = tail_d

                cnt2 = jnp.where(cnt2 >= 64, cnt2 - 64, cnt2)
                return cnt2

            return lax.fori_loop(0, _KCH // 16, grp, cnt)

        cnt = lax.fori_loop(0, nit, chunk, jnp.int32(0))
        dummy_src = jnp.full((16,), cbase + npad - 8, jnp.int32) + lanes % 8
        zero16 = jnp.zeros((16,), jnp.int32)

        def pad16(j, carry):
            srcf[pl.ds(cnt + j * 16, 16)] = dummy_src
            dstf[pl.ds(cnt + j * 16, 16)] = zero16
            return carry

        lax.fori_loop(0, 4, pad16, 0)

        @pl.when(cnt > 0)
        def _():
            flush()

        q = c * 16 + s
        pltpu.sync_copy(accp.at[pl.ds(128, rpt * 128)],
                        acc_h.at[pl.ds(q * rpt * 128, rpt * 128)])

    return k(g2, srcp, dstp, zacc, lut)


def _sc_pair(aarr, barr, psp, pdp, pw2flat):
    npad, hid = aarr.shape
    ppad = psp.shape[0]
    ppt = ppad // 32
    nit = ppt // _CHP
    nj = hid // 16
    mesh = plsc.VectorSubcoreMesh(core_axis_name="c", subcore_axis_name="s")

    @functools.partial(
        pl.kernel, mesh=mesh,
        out_type=jax.ShapeDtypeStruct((ppad,), jnp.float32),
        scratch_types=[pltpu.VMEM((_CHP,), jnp.int32),
                       pltpu.VMEM((_CHP,), jnp.int32),
                       pltpu.VMEM((_CHP, hid), jnp.float32),
                       pltpu.VMEM((_CHP, hid), jnp.float32),
                       pltpu.VMEM((_CHP,), jnp.float32),
                       pltpu.VMEM((hid,), jnp.float32),
                       pltpu.SemaphoreType.DMA,
                       pltpu.SemaphoreType.DMA],
    )
    def k(a_h, b_h, ps_h, pd_h, pw2_h, out_h,
          psv, pdv, ra, rb, sv, pw2v, semA, semB):
        c = lax.axis_index("c")
        s = lax.axis_index("s")
        w = s * 2 + c
        pltpu.sync_copy(pw2_h, pw2v)

        def chunk(it, carry):
            base = w * ppt + it * _CHP
            pltpu.sync_copy(ps_h.at[pl.ds(base, _CHP)], psv)
            pltpu.sync_copy(pd_h.at[pl.ds(base, _CHP)], pdv)
            cpa = pltpu.async_copy(a_h.at[psv], ra, semA)
            cpb = pltpu.async_copy(b_h.at[pdv], rb, semB)
            cpa.wait()
            cpb.wait()

            lanes = lax.broadcasted_iota(jnp.int32, (16,), 0)
            perms = [jnp.bitwise_xor(lanes, sh) for sh in (8, 4, 2, 1)]

            def group(gi, carry2):
                i0 = gi * 16
                svec = jnp.zeros((16,), jnp.float32)
                for kk in range(16):
                    i = i0 + kk
                    acc = jnp.zeros((16,), jnp.float32)
                    for j in range(nj):
                        va = ra[i, pl.ds(j * 16, 16)]
                        vb = rb[i, pl.ds(j * 16, 16)]
                        acc = acc + jnp.maximum(va + vb, 0.0) \
                            * pw2v[pl.ds(j * 16, 16)]
                    for perm in perms:
                        acc = acc + _lane_shuffle(acc, perm)
                    svec = jnp.where(lanes == kk, acc, svec)
                sv[pl.ds(gi * 16, 16)] = svec
                return carry2

            lax.fori_loop(0, _CHP // 16, group, 0)
            pltpu.sync_copy(sv, out_h.at[pl.ds(base, _CHP)])
            return carry

        lax.fori_loop(0, nit, chunk, 0)

    return k(aarr, barr, psp, pdp, pw2flat)


def kernel(x, edge_index, pairs, W0, b0, W1, b1, W2, b2, PW1, Pb1, PW2, Pb2):
    N, din = x.shape
    hid = W0.shape[1]
    E = edge_index.shape[1]
    P = pairs.shape[0]

    npad = _ceil_to(N + 1, 2 * _RB)
    epad = _ceil_to(E, 32 * _KCH)
    ppad = _ceil_to(P, 32 * _CHP)
    half = npad // 2

    src = edge_index[0].astype(jnp.int32)
    dst = edge_index[1].astype(jnp.int32)
    ndum = npad - N
    if epad > E:
        fill = N + jnp.arange(epad - E, dtype=jnp.int32) % ndum
        src = jnp.concatenate([src, fill])
        dst = jnp.concatenate([dst, fill])
    ps = pairs[:, 0].astype(jnp.int32)
    pd = pairs[:, 1].astype(jnp.int32)
    if ppad > P:
        pfill = N + jnp.arange(ppad - P, dtype=jnp.int32) % ndum
        ps = jnp.concatenate([ps, pfill])
        pd = jnp.concatenate([pd, pfill])

    xp = jnp.pad(x, ((0, npad - N), (0, 0)))
    zacc8 = jnp.zeros(((npad + 2) * 8,), jnp.float32)
    zacc = jnp.zeros(((npad // 16 + 1) * 128,), jnp.float32)
    lut_np = np.zeros((256, 128), np.int32)
    for m in range(256):
        pos = [i for i in range(8) if (m >> i) & 1]
        lut_np[m, :len(pos)] = pos
        lut_np[m, 8] = len(pos)
    lut = jnp.asarray(lut_np)

    def scat(g2_):
        return _sc_scat(g2_, src, dst, zacc, lut, npad)\
            .reshape(2 * npad, 128)
    b0r = b0.reshape(1, hid)
    b1r = b1.reshape(1, hid)
    b2r = b2.reshape(1, hid)
    pb1r = Pb1.reshape(1, hid)
    pw1a = PW1[:hid]
    pw1b = PW1[hid:]
    pw2flat = PW2[:, 0]

    deg32r = _sc_deg(dst, zacc8, npad).reshape(32 * npad, 8)
    g2, dinv8 = _tc_first(xp, W0, deg32r, N)
    acc2 = scat(g2)
    g2 = _tc_mid(acc2, g2, dinv8, b0r, W1)
    acc2 = scat(g2)
    g2 = _tc_mid(acc2, g2, dinv8, b1r, W2)
    acc2 = scat(g2)
    aarr, barr = _tc_fin(acc2, g2, dinv8, b2r, pw1a, pw1b, pb1r)
    scores = _sc_pair(aarr, barr, ps, pd, pw2flat)
    return scores[:P] + Pb2[0]


# speedup vs baseline: 1.8860x; 1.1174x over previous
import functools

import jax
import numpy as np
import jax.numpy as jnp
from jax import lax
from jax.experimental import pallas as pl
from jax.experimental.pallas import tpu as pltpu
from jax.experimental.pallas import tpu_sc as plsc

_RB = 512
_KCH = 256
_CHP = 128


def _ceil_to(a, m):
    return (a + m - 1) // m * m


def _lane_shuffle(v, perm):
    dnums = lax.GatherDimensionNumbers(
        offset_dims=(), collapsed_slice_dims=(0,), start_index_map=(0,))
    return lax.gather(v, perm[:, None], dnums, (1,),
                      mode=lax.GatherScatterMode.PROMISE_IN_BOUNDS)


def _tc_first(xp, w0, deg32r, n_real):
    npad, din = xp.shape
    hid = w0.shape[1]
    hh = hid // 2
    nb = npad // _RB

    def body(x_ref, w_ref, *rest):
        dg = rest[:32]
        g_ref, dv_ref = rest[32], rest[33]
        i = pl.program_id(0)
        rows = lax.broadcasted_iota(jnp.int32, (_RB, 1), 0) + i * _RB
        deg = dg[0][:, 0:1] + 1.0
        for k in range(1, 32):
            deg = deg + dg[k][:, 0:1]
        dinv = jnp.where(rows < n_real, lax.rsqrt(deg), 0.0)
        g_ref[...] = jnp.dot(x_ref[...], w_ref[...],
                             preferred_element_type=jnp.float32) * dinv
        dv_ref[...] = jnp.broadcast_to(dinv, (_RB, 8))

    deg_specs = [pl.BlockSpec((_RB, 8), (lambda i, h, k=k: (k * nb + i, 0)))
                 for k in range(32)]
    return pl.pallas_call(
        body,
        grid=(nb, 2),
        in_specs=[pl.BlockSpec((_RB, din), lambda i, h: (i, 0)),
                  pl.BlockSpec((din, hh), lambda i, h: (0, h))] + deg_specs,
        out_specs=[pl.BlockSpec((_RB, hh), lambda i, h: (h * nb + i, 0)),
                   pl.BlockSpec((_RB, 8), lambda i, h: (i, 0))],
        out_shape=[jax.ShapeDtypeStruct((2 * npad, hh), jnp.float32),
                   jax.ShapeDtypeStruct((npad, 8), jnp.float32)],
    )(xp, w0, *([deg32r] * 32))


def _tc_mid(acc2, g2, dinv8, brow, w):
    npad2, hh = g2.shape
    npad = npad2 // 2
    hid = 2 * hh
    nb = npad // _RB

    def body(aa, ab, ga, gb, dv, b_r, w_r, go):
        dinv = dv[:, 0:1]
        acc = jnp.concatenate([aa[...] + ga[...], ab[...] + gb[...]], axis=1)
        h = jnp.maximum(acc * dinv + b_r[...], 0.0)
        go[...] = jnp.dot(h, w_r[...],
                          preferred_element_type=jnp.float32) * dinv

    return pl.pallas_call(
        body,
        grid=(nb, 2),
        in_specs=[pl.BlockSpec((_RB, hh), lambda i, h: (i, 0)),
                  pl.BlockSpec((_RB, hh), lambda i, h: (nb + i, 0)),
                  pl.BlockSpec((_RB, hh), lambda i, h: (i, 0)),
                  pl.BlockSpec((_RB, hh), lambda i, h: (nb + i, 0)),
                  pl.BlockSpec((_RB, 8), lambda i, h: (i, 0)),
                  pl.BlockSpec((1, hid), lambda i, h: (0, 0)),
                  pl.BlockSpec((hid, hh), lambda i, h: (0, h))],
        out_specs=pl.BlockSpec((_RB, hh), lambda i, h: (h * nb + i, 0)),
        out_shape=jax.ShapeDtypeStruct((2 * npad, hh), jnp.float32),
    )(acc2, acc2, g2, g2, dinv8, brow, w)


def _tc_fin(acc2, g2, dinv8, brow, pw1a, pw1b, pb1row):
    npad2, hh = g2.shape
    npad = npad2 // 2
    hid = 2 * hh
    nb = npad // _RB

    def body(aa, ab, ga, gb, dv, b_r, wa_r, wb_r, pb_r, ao, bo):
        dinv = dv[:, 0:1]
        acc = jnp.concatenate([aa[...] + ga[...], ab[...] + gb[...]], axis=1)
        h = jnp.maximum(acc * dinv + b_r[...], 0.0)
        ao[...] = jnp.dot(h, wa_r[...],
                          preferred_element_type=jnp.float32) + pb_r[...]
        bo[...] = jnp.dot(h, wb_r[...], preferred_element_type=jnp.float32)

    return pl.pallas_call(
        body,
        grid=(nb,),
        in_specs=[pl.BlockSpec((_RB, hh), lambda i: (i, 0)),
                  pl.BlockSpec((_RB, hh), lambda i: (nb + i, 0)),
                  pl.BlockSpec((_RB, hh), lambda i: (i, 0)),
                  pl.BlockSpec((_RB, hh), lambda i: (nb + i, 0)),
                  pl.BlockSpec((_RB, 8), lambda i: (i, 0)),
                  pl.BlockSpec((1, hid), lambda i: (0, 0)),
                  pl.BlockSpec((hid, hid), lambda i: (0, 0)),
                  pl.BlockSpec((hid, hid), lambda i: (0, 0)),
                  pl.BlockSpec((1, hid), lambda i: (0, 0))],
        out_specs=[pl.BlockSpec((_RB, hid), lambda i: (i, 0)),
                   pl.BlockSpec((_RB, hid), lambda i: (i, 0))],
        out_shape=[jax.ShapeDtypeStruct((npad, hid), jnp.float32),
                   jax.ShapeDtypeStruct((npad, hid), jnp.float32)],
    )(acc2, acc2, g2, g2, dinv8, brow, pw1a, pw1b, pb1row)


def _sc_deg(dstp, zacc8, npad):
    epad = dstp.shape[0]
    ept = epad // 32
    nit = ept // _KCH
    nrow8 = (npad + 2) * 8
    rpt8 = (npad // 16) * 8
    mesh = plsc.VectorSubcoreMesh(core_axis_name="c", subcore_axis_name="s")

    @functools.partial(
        pl.kernel, mesh=mesh,
        out_type=jax.ShapeDtypeStruct((32 * npad * 8,), jnp.float32),
        scratch_types=[pltpu.VMEM((_KCH,), jnp.int32),
                       pltpu.VMEM((nrow8,), jnp.float32)],
    )
    def k(dst_h, z_h, deg_h, dstv, accp):
        c = lax.axis_index("c")
        s = lax.axis_index("s")
        w = s * 2 + c
        pltpu.sync_copy(z_h, accp)
        lanes = lax.broadcasted_iota(jnp.int32, (16,), 0)
        v_lo = jnp.where(lanes < 8, 1.0, 0.0).astype(jnp.float32)
        v_hi = jnp.where(lanes < 8, 0.0, 1.0).astype(jnp.float32)

        def chunk(it, carry):
            base = w * ept + it * _KCH
            pltpu.sync_copy(dst_h.at[pl.ds(base, _KCH)], dstv)

            def grp(i, carry2):
                dv16 = dstv[pl.ds(i * 16, 16)]
                lo16 = (dv16 + 1) * 8
                hi16 = dv16 * 8
                for kk in range(0, 16, 2):
                    plsc.addupdate(accp.at[pl.ds(lo16[kk], 16)], v_lo)
                    plsc.addupdate(accp.at[pl.ds(hi16[kk + 1], 16)], v_hi)
                return carry2

            lax.fori_loop(0, _KCH // 16, grp, 0)
            return carry

        lax.fori_loop(0, nit, chunk, 0)
        pltpu.sync_copy(accp.at[pl.ds(8, npad * 8)],
                        deg_h.at[pl.ds(w * npad * 8, npad * 8)])

    return k(dstp, zacc8)


def _sc_scat(g2, srcp, dstp, zacc, lut, npad):
    epad = dstp.shape[0]
    nit = epad // _KCH
    rpt = npad // 16
    mesh = plsc.VectorSubcoreMesh(core_axis_name="c", subcore_axis_name="s")

    @functools.partial(
        pl.kernel, mesh=mesh,
        out_type=jax.ShapeDtypeStruct((32 * rpt * 128,), jnp.float32),
        scratch_types=[pltpu.VMEM((_KCH,), jnp.int32),
                       pltpu.VMEM((_KCH,), jnp.int32),
                       pltpu.VMEM((192,), jnp.int32),
                       pltpu.VMEM((192,), jnp.int32),
                       pltpu.VMEM((256, 128), jnp.int32),
                       pltpu.VMEM((64, 128), jnp.float32),
                       pltpu.VMEM(((rpt + 1) * 128,), jnp.float32),
                       pltpu.SemaphoreType.DMA],
    )
    def k(g_h, src_h, dst_h, z_h, lut_h, acc_h, srcv, dstv, srcf, dstf,
          lutv, rows, accp, sem):
        c = lax.axis_index("c")
        s = lax.axis_index("s")
        cbase = c * npad
        nodebase = s * rpt
        pltpu.sync_copy(z_h, accp)
        pltpu.sync_copy(lut_h, lutv)
        lanes = lax.broadcasted_iota(jnp.int32, (16,), 0)
        pow2 = jnp.left_shift(1, lanes % 8)
        perms_h = [jnp.bitwise_xor(lanes, sh) for sh in (4, 2, 1)]

        def flush():
            cp = pltpu.async_copy(g_h.at[srcf.at[pl.ds(0, 64)]], rows, sem)
            cp.wait()

            def grp(i, carry2):
                i0 = i * 16
                dl16 = dstf[pl.ds(i0, 16)] * 128
                for kk in range(16):
                    base2 = dl16[kk]
                    for j in range(8):
                        plsc.addupdate(
                            accp.at[pl.ds(base2 + j * 16, 16)],
                            rows[i0 + kk, pl.ds(j * 16, 16)])
                return carry2

            lax.fori_loop(0, 4, grp, 0)

        def chunk(it, cnt):
            base = it * _KCH
            pltpu.sync_copy(src_h.at[pl.ds(base, _KCH)], srcv)
            pltpu.sync_copy(dst_h.at[pl.ds(base, _KCH)], dstv)

            def grp(i, cnt2):
                i0 = i * 64
                res = []
                for u in range(4):
                    dv16 = dstv[pl.ds(i0 + u * 16, 16)]
                    sv16 = srcv[pl.ds(i0 + u * 16, 16)] + cbase
                    dloc = dv16 - nodebase + 1
                    own = jnp.logical_and(dloc >= 1, dloc <= rpt)
                    mv = jnp.where(own, pow2, 0)
                    for perm in perms_h:
                        mv = mv + _lane_shuffle(mv, perm)
                    lr0 = lutv[mv[0], pl.ds(0, 16)]
                    lr1 = lutv[mv[8], pl.ds(0, 16)]
                    perm1 = lr1 + 8
                    res.append((_lane_shuffle(sv16, lr0),
                                _lane_shuffle(dloc, lr0), lr0[8],
                                _lane_shuffle(sv16, perm1),
                                _lane_shuffle(dloc, perm1), lr1[8]))
                for cs0, cd0, p0, cs1, cd1, p1 in res:
                    srcf[pl.ds(cnt2, 16)] = cs0
                    dstf[pl.ds(cnt2, 16)] = cd0
                    cnt2 = cnt2 + p0
                    srcf[pl.ds(cnt2, 16)] = cs1
                    dstf[pl.ds(cnt2, 16)] = cd1
                    cnt2 = cnt2 + p1

                    @pl.when(cnt2 >= 64)
                    def _():
                        flush()
                        for t in range(2):
                            tail_s = srcf[pl.ds(64 + t * 16, 16)]
                            tail_d = dstf[pl.ds(64 + t * 16, 16)]
                            srcf[pl.ds(t * 16, 16)] = tail_s
                            dstf[pl.ds(t * 16, 16)] = tail_d

                    cnt2 = jnp.where(cnt2 >= 64, cnt2 - 64, cnt2)
                return cnt2

            return lax.fori_loop(0, _KCH // 64, grp, cnt)

        cnt = lax.fori_loop(0, nit, chunk, jnp.int32(0))
        dummy_src = jnp.full((16,), cbase + npad - 8, jnp.int32) + lanes % 8
        zero16 = jnp.zeros((16,), jnp.int32)

        def pad16(j, carry):
            srcf[pl.ds(cnt + j * 16, 16)] = dummy_src
            dstf[pl.ds(cnt + j * 16, 16)] = zero16
            return carry

        lax.fori_loop(0, 4, pad16, 0)

        @pl.when(cnt > 0)
        def _():
            flush()

        q = c * 16 + s
        pltpu.sync_copy(accp.at[pl.ds(128, rpt * 128)],
                        acc_h.at[pl.ds(q * rpt * 128, rpt * 128)])

    return k(g2, srcp, dstp, zacc, lut)


def _sc_pair(aarr, barr, psp, pdp, pw2flat):
    npad, hid = aarr.shape
    ppad = psp.shape[0]
    ppt = ppad // 32
    nit = ppt // _CHP
    nj = hid // 16
    mesh = plsc.VectorSubcoreMesh(core_axis_name="c", subcore_axis_name="s")

    @functools.partial(
        pl.kernel, mesh=mesh,
        out_type=jax.ShapeDtypeStruct((ppad,), jnp.float32),
        scratch_types=[pltpu.VMEM((_CHP,), jnp.int32),
                       pltpu.VMEM((_CHP,), jnp.int32),
                       pltpu.VMEM((_CHP, hid), jnp.float32),
                       pltpu.VMEM((_CHP, hid), jnp.float32),
                       pltpu.VMEM((_CHP,), jnp.float32),
                       pltpu.VMEM((hid,), jnp.float32),
                       pltpu.SemaphoreType.DMA,
                       pltpu.SemaphoreType.DMA],
    )
    def k(a_h, b_h, ps_h, pd_h, pw2_h, out_h,
          psv, pdv, ra, rb, sv, pw2v, semA, semB):
        c = lax.axis_index("c")
        s = lax.axis_index("s")
        w = s * 2 + c
        pltpu.sync_copy(pw2_h, pw2v)

        def chunk(it, carry):
            base = w * ppt + it * _CHP
            pltpu.sync_copy(ps_h.at[pl.ds(base, _CHP)], psv)
            pltpu.sync_copy(pd_h.at[pl.ds(base, _CHP)], pdv)
            cpa = pltpu.async_copy(a_h.at[psv], ra, semA)
            cpb = pltpu.async_copy(b_h.at[pdv], rb, semB)
            cpa.wait()
            cpb.wait()

            lanes = lax.broadcasted_iota(jnp.int32, (16,), 0)
            perms = [jnp.bitwise_xor(lanes, sh) for sh in (8, 4, 2, 1)]

            def group(gi, carry2):
                i0 = gi * 16
                svec = jnp.zeros((16,), jnp.float32)
                for kk in range(16):
                    i = i0 + kk
                    acc = jnp.zeros((16,), jnp.float32)
                    for j in range(nj):
                        va = ra[i, pl.ds(j * 16, 16)]
                        vb = rb[i, pl.ds(j * 16, 16)]
                        acc = acc + jnp.maximum(va + vb, 0.0) \
                            * pw2v[pl.ds(j * 16, 16)]
                    for perm in perms:
                        acc = acc + _lane_shuffle(acc, perm)
                    svec = jnp.where(lanes == kk, acc, svec)
                sv[pl.ds(gi * 16, 16)] = svec
                return carry2

            lax.fori_loop(0, _CHP // 16, group, 0)
            pltpu.sync_copy(sv, out_h.at[pl.ds(base, _CHP)])
            return carry

        lax.fori_loop(0, nit, chunk, 0)

    return k(aarr, barr, psp, pdp, pw2flat)


def kernel(x, edge_index, pairs, W0, b0, W1, b1, W2, b2, PW1, Pb1, PW2, Pb2):
    N, din = x.shape
    hid = W0.shape[1]
    E = edge_index.shape[1]
    P = pairs.shape[0]

    npad = _ceil_to(N + 1, 2 * _RB)
    epad = _ceil_to(E, 32 * _KCH)
    ppad = _ceil_to(P, 32 * _CHP)
    half = npad // 2

    src = edge_index[0].astype(jnp.int32)
    dst = edge_index[1].astype(jnp.int32)
    ndum = npad - N
    if epad > E:
        fill = N + jnp.arange(epad - E, dtype=jnp.int32) % ndum
        src = jnp.concatenate([src, fill])
        dst = jnp.concatenate([dst, fill])
    ps = pairs[:, 0].astype(jnp.int32)
    pd = pairs[:, 1].astype(jnp.int32)
    if ppad > P:
        pfill = N + jnp.arange(ppad - P, dtype=jnp.int32) % ndum
        ps = jnp.concatenate([ps, pfill])
        pd = jnp.concatenate([pd, pfill])

    xp = jnp.pad(x, ((0, npad - N), (0, 0)))
    zacc8 = jnp.zeros(((npad + 2) * 8,), jnp.float32)
    zacc = jnp.zeros(((npad // 16 + 1) * 128,), jnp.float32)
    lut_np = np.zeros((256, 128), np.int32)
    for m in range(256):
        pos = [i for i in range(8) if (m >> i) & 1]
        lut_np[m, :len(pos)] = pos
        lut_np[m, 8] = len(pos)
    lut = jnp.asarray(lut_np)

    def scat(g2_):
        return _sc_scat(g2_, src, dst, zacc, lut, npad)\
            .reshape(2 * npad, 128)
    b0r = b0.reshape(1, hid)
    b1r = b1.reshape(1, hid)
    b2r = b2.reshape(1, hid)
    pb1r = Pb1.reshape(1, hid)
    pw1a = PW1[:hid]
    pw1b = PW1[hid:]
    pw2flat = PW2[:, 0]

    deg32r = _sc_deg(dst, zacc8, npad).reshape(32 * npad, 8)
    g2, dinv8 = _tc_first(xp, W0, deg32r, N)
    acc2 = scat(g2)
    g2 = _tc_mid(acc2, g2, dinv8, b0r, W1)
    acc2 = scat(g2)
    g2 = _tc_mid(acc2, g2, dinv8, b1r, W2)
    acc2 = scat(g2)
    aarr, barr = _tc_fin(acc2, g2, dinv8, b2r, pw1a, pw1b, pb1r)
    scores = _sc_pair(aarr, barr, ps, pd, pw2flat)
    return scores[:P] + Pb2[0]


# speedup vs baseline: 2.6779x; 1.4199x over previous
import functools

import jax
import numpy as np
import jax.numpy as jnp
from jax import lax
from jax.experimental import pallas as pl
from jax.experimental.pallas import tpu as pltpu
from jax.experimental.pallas import tpu_sc as plsc

_RB = 512
_KCH = 1024
_CHP = 128


def _ceil_to(a, m):
    return (a + m - 1) // m * m


def _lane_shuffle(v, perm):
    dnums = lax.GatherDimensionNumbers(
        offset_dims=(), collapsed_slice_dims=(0,), start_index_map=(0,))
    return lax.gather(v, perm[:, None], dnums, (1,),
                      mode=lax.GatherScatterMode.PROMISE_IN_BOUNDS)


def _tc_first(xp, w0, deg32r, n_real):
    npad, din = xp.shape
    hid = w0.shape[1]
    hh = hid // 2
    nb = npad // _RB

    def body(x_ref, w_ref, *rest):
        dg = rest[:32]
        g_ref, dv_ref = rest[32], rest[33]
        i = pl.program_id(0)
        rows = lax.broadcasted_iota(jnp.int32, (_RB, 1), 0) + i * _RB
        deg = dg[0][:, 0:1] + 1.0
        for k in range(1, 32):
            deg = deg + dg[k][:, 0:1]
        dinv = jnp.where(rows < n_real, lax.rsqrt(deg), 0.0)
        g_ref[...] = jnp.dot(x_ref[...], w_ref[...],
                             preferred_element_type=jnp.float32) * dinv
        dv_ref[...] = jnp.broadcast_to(dinv, (_RB, 8))

    deg_specs = [pl.BlockSpec((_RB, 8), (lambda i, h, k=k: (k * nb + i, 0)))
                 for k in range(32)]
    return pl.pallas_call(
        body,
        grid=(nb, 2),
        in_specs=[pl.BlockSpec((_RB, din), lambda i, h: (i, 0)),
                  pl.BlockSpec((din, hh), lambda i, h: (0, h))] + deg_specs,
        out_specs=[pl.BlockSpec((_RB, hh), lambda i, h: (h * nb + i, 0)),
                   pl.BlockSpec((_RB, 8), lambda i, h: (i, 0))],
        out_shape=[jax.ShapeDtypeStruct((2 * npad, hh), jnp.float32),
                   jax.ShapeDtypeStruct((npad, 8), jnp.float32)],
    )(xp, w0, *([deg32r] * 32))


def _tc_mid(acc2, g2, dinv8, brow, w):
    npad2, hh = g2.shape
    npad = npad2 // 2
    hid = 2 * hh
    nb = npad // _RB

    def body(aa, ab, ga, gb, dv, b_r, w_r, go):
        dinv = dv[:, 0:1]
        acc = jnp.concatenate([aa[...] + ga[...], ab[...] + gb[...]], axis=1)
        h = jnp.maximum(acc * dinv + b_r[...], 0.0)
        go[...] = jnp.dot(h, w_r[...],
                          preferred_element_type=jnp.float32) * dinv

    return pl.pallas_call(
        body,
        grid=(nb, 2),
        in_specs=[pl.BlockSpec((_RB, hh), lambda i, h: (i, 0)),
                  pl.BlockSpec((_RB, hh), lambda i, h: (nb + i, 0)),
                  pl.BlockSpec((_RB, hh), lambda i, h: (i, 0)),
                  pl.BlockSpec((_RB, hh), lambda i, h: (nb + i, 0)),
                  pl.BlockSpec((_RB, 8), lambda i, h: (i, 0)),
                  pl.BlockSpec((1, hid), lambda i, h: (0, 0)),
                  pl.BlockSpec((hid, hh), lambda i, h: (0, h))],
        out_specs=pl.BlockSpec((_RB, hh), lambda i, h: (h * nb + i, 0)),
        out_shape=jax.ShapeDtypeStruct((2 * npad, hh), jnp.float32),
    )(acc2, acc2, g2, g2, dinv8, brow, w)


def _tc_fin(acc2, g2, dinv8, brow, pw1a, pw1b, pb1row):
    npad2, hh = g2.shape
    npad = npad2 // 2
    hid = 2 * hh
    nb = npad // _RB

    def body(aa, ab, ga, gb, dv, b_r, wa_r, wb_r, pb_r, ao, bo):
        dinv = dv[:, 0:1]
        acc = jnp.concatenate([aa[...] + ga[...], ab[...] + gb[...]], axis=1)
        h = jnp.maximum(acc * dinv + b_r[...], 0.0)
        ao[...] = jnp.dot(h, wa_r[...],
                          preferred_element_type=jnp.float32) + pb_r[...]
        bo[...] = jnp.dot(h, wb_r[...], preferred_element_type=jnp.float32)

    return pl.pallas_call(
        body,
        grid=(nb,),
        in_specs=[pl.BlockSpec((_RB, hh), lambda i: (i, 0)),
                  pl.BlockSpec((_RB, hh), lambda i: (nb + i, 0)),
                  pl.BlockSpec((_RB, hh), lambda i: (i, 0)),
                  pl.BlockSpec((_RB, hh), lambda i: (nb + i, 0)),
                  pl.BlockSpec((_RB, 8), lambda i: (i, 0)),
                  pl.BlockSpec((1, hid), lambda i: (0, 0)),
                  pl.BlockSpec((hid, hid), lambda i: (0, 0)),
                  pl.BlockSpec((hid, hid), lambda i: (0, 0)),
                  pl.BlockSpec((1, hid), lambda i: (0, 0))],
        out_specs=[pl.BlockSpec((_RB, hid), lambda i: (i, 0)),
                   pl.BlockSpec((_RB, hid), lambda i: (i, 0))],
        out_shape=[jax.ShapeDtypeStruct((npad, hid), jnp.float32),
                   jax.ShapeDtypeStruct((npad, hid), jnp.float32)],
    )(acc2, acc2, g2, g2, dinv8, brow, pw1a, pw1b, pb1row)


def _sc_deg(dstp, zacc8, npad):
    epad = dstp.shape[0]
    ept = epad // 32
    nit = ept // _KCH
    nrow8 = (npad + 2) * 8
    rpt8 = (npad // 16) * 8
    mesh = plsc.VectorSubcoreMesh(core_axis_name="c", subcore_axis_name="s")

    @functools.partial(
        pl.kernel, mesh=mesh,
        out_type=jax.ShapeDtypeStruct((32 * npad * 8,), jnp.float32),
        scratch_types=[pltpu.VMEM((_KCH,), jnp.int32),
                       pltpu.VMEM((nrow8,), jnp.float32)],
    )
    def k(dst_h, z_h, deg_h, dstv, accp):
        c = lax.axis_index("c")
        s = lax.axis_index("s")
        w = s * 2 + c
        pltpu.sync_copy(z_h, accp)
        lanes = lax.broadcasted_iota(jnp.int32, (16,), 0)
        v_lo = jnp.where(lanes < 8, 1.0, 0.0).astype(jnp.float32)
        v_hi = jnp.where(lanes < 8, 0.0, 1.0).astype(jnp.float32)

        def chunk(it, carry):
            base = w * ept + it * _KCH
            pltpu.sync_copy(dst_h.at[pl.ds(base, _KCH)], dstv)

            def grp(i, carry2):
                dv16 = dstv[pl.ds(i * 16, 16)]
                lo16 = (dv16 + 1) * 8
                hi16 = dv16 * 8
                for kk in range(0, 16, 2):
                    plsc.addupdate(accp.at[pl.ds(lo16[kk], 16)], v_lo)
                    plsc.addupdate(accp.at[pl.ds(hi16[kk + 1], 16)], v_hi)
                return carry2

            lax.fori_loop(0, _KCH // 16, grp, 0)
            return carry

        lax.fori_loop(0, nit, chunk, 0)
        pltpu.sync_copy(accp.at[pl.ds(8, npad * 8)],
                        deg_h.at[pl.ds(w * npad * 8, npad * 8)])

    return k(dstp, zacc8)


def _sc_scat(g2, srcp, dstp, zacc, lut, npad):
    epad = dstp.shape[0]
    nit = epad // _KCH
    rpt = npad // 16
    mesh = plsc.VectorSubcoreMesh(core_axis_name="c", subcore_axis_name="s")

    @functools.partial(
        pl.kernel, mesh=mesh,
        out_type=jax.ShapeDtypeStruct((32 * rpt * 128,), jnp.float32),
        scratch_types=[pltpu.VMEM((_KCH,), jnp.int32),
                       pltpu.VMEM((_KCH,), jnp.int32),
                       pltpu.VMEM((192,), jnp.int32),
                       pltpu.VMEM((192,), jnp.int32),
                       pltpu.VMEM((256, 128), jnp.int32),
                       pltpu.VMEM((64, 128), jnp.float32),
                       pltpu.VMEM(((rpt + 1) * 128,), jnp.float32),
                       pltpu.SemaphoreType.DMA],
    )
    def k(g_h, src_h, dst_h, z_h, lut_h, acc_h, srcv, dstv, srcf, dstf,
          lutv, rows, accp, sem):
        c = lax.axis_index("c")
        s = lax.axis_index("s")
        cbase = c * npad
        nodebase = s * rpt
        pltpu.sync_copy(z_h, accp)
        pltpu.sync_copy(lut_h, lutv)
        lanes = lax.broadcasted_iota(jnp.int32, (16,), 0)
        pow2 = jnp.left_shift(1, lanes % 8)
        perms_h = [jnp.bitwise_xor(lanes, sh) for sh in (4, 2, 1)]

        def flush():
            cp = pltpu.async_copy(g_h.at[srcf.at[pl.ds(0, 64)]], rows, sem)
            cp.wait()

            def grp(i, carry2):
                i0 = i * 16
                dl16 = dstf[pl.ds(i0, 16)] * 128
                for kk in range(16):
                    base2 = dl16[kk]
                    for j in range(8):
                        plsc.addupdate(
                            accp.at[pl.ds(base2 + j * 16, 16)],
                            rows[i0 + kk, pl.ds(j * 16, 16)])
                return carry2

            lax.fori_loop(0, 4, grp, 0)

        def chunk(it, cnt):
            base = it * _KCH
            pltpu.sync_copy(src_h.at[pl.ds(base, _KCH)], srcv)
            pltpu.sync_copy(dst_h.at[pl.ds(base, _KCH)], dstv)

            def grp(i, cnt2):
                i0 = i * 64
                res = []
                for u in range(4):
                    dv16 = dstv[pl.ds(i0 + u * 16, 16)]
                    sv16 = srcv[pl.ds(i0 + u * 16, 16)] + cbase
                    dloc = dv16 - nodebase + 1
                    own = jnp.logical_and(dloc >= 1, dloc <= rpt)
                    mv = jnp.where(own, pow2, 0)
                    for perm in perms_h:
                        mv = mv + _lane_shuffle(mv, perm)
                    lr0 = lutv[mv[0], pl.ds(0, 16)]
                    lr1 = lutv[mv[8], pl.ds(0, 16)]
                    perm1 = lr1 + 8
                    res.append((_lane_shuffle(sv16, lr0),
                                _lane_shuffle(dloc, lr0), lr0[8],
                                _lane_shuffle(sv16, perm1),
                                _lane_shuffle(dloc, perm1), lr1[8]))
                for cs0, cd0, p0, cs1, cd1, p1 in res:
                    srcf[pl.ds(cnt2, 16)] = cs0
                    dstf[pl.ds(cnt2, 16)] = cd0
                    cnt2 = cnt2 + p0
                    srcf[pl.ds(cnt2, 16)] = cs1
                    dstf[pl.ds(cnt2, 16)] = cd1
                    cnt2 = cnt2 + p1

                    @pl.when(cnt2 >= 64)
                    def _():
                        flush()
                        for t in range(2):
                            tail_s = srcf[pl.ds(64 + t * 16, 16)]
                            tail_d = dstf[pl.ds(64 + t * 16, 16)]
                            srcf[pl.ds(t * 16, 16)] = tail_s
                            dstf[pl.ds(t * 16, 16)] = tail_d

                    cnt2 = jnp.where(cnt2 >= 64, cnt2 - 64, cnt2)
                return cnt2

            return lax.fori_loop(0, _KCH // 64, grp, cnt)

        cnt = lax.fori_loop(0, nit, chunk, jnp.int32(0))
        dummy_src = jnp.full((16,), cbase + npad - 8, jnp.int32) + lanes % 8
        zero16 = jnp.zeros((16,), jnp.int32)

        def pad16(j, carry):
            srcf[pl.ds(cnt + j * 16, 16)] = dummy_src
            dstf[pl.ds(cnt + j * 16, 16)] = zero16
            return carry

        lax.fori_loop(0, 4, pad16, 0)

        @pl.when(cnt > 0)
        def _():
            flush()

        q = c * 16 + s
        pltpu.sync_copy(accp.at[pl.ds(128, rpt * 128)],
                        acc_h.at[pl.ds(q * rpt * 128, rpt * 128)])

    return k(g2, srcp, dstp, zacc, lut)


def _sc_pair(aarr, barr, psp, pdp, pw2flat):
    npad, hid = aarr.shape
    ppad = psp.shape[0]
    ppt = ppad // 32
    nit = ppt // _CHP
    nj = hid // 16
    mesh = plsc.VectorSubcoreMesh(core_axis_name="c", subcore_axis_name="s")

    @functools.partial(
        pl.kernel, mesh=mesh,
        out_type=jax.ShapeDtypeStruct((ppad,), jnp.float32),
        scratch_types=[pltpu.VMEM((_CHP,), jnp.int32),
                       pltpu.VMEM((_CHP,), jnp.int32),
                       pltpu.VMEM((_CHP, hid), jnp.float32),
                       pltpu.VMEM((_CHP, hid), jnp.float32),
                       pltpu.VMEM((_CHP,), jnp.float32),
                       pltpu.VMEM((hid,), jnp.float32),
                       pltpu.SemaphoreType.DMA,
                       pltpu.SemaphoreType.DMA],
    )
    def k(a_h, b_h, ps_h, pd_h, pw2_h, out_h,
          psv, pdv, ra, rb, sv, pw2v, semA, semB):
        c = lax.axis_index("c")
        s = lax.axis_index("s")
        w = s * 2 + c
        pltpu.sync_copy(pw2_h, pw2v)

        def chunk(it, carry):
            base = w * ppt + it * _CHP
            pltpu.sync_copy(ps_h.at[pl.ds(base, _CHP)], psv)
            pltpu.sync_copy(pd_h.at[pl.ds(base, _CHP)], pdv)
            cpa = pltpu.async_copy(a_h.at[psv], ra, semA)
            cpb = pltpu.async_copy(b_h.at[pdv], rb, semB)
            cpa.wait()
            cpb.wait()

            lanes = lax.broadcasted_iota(jnp.int32, (16,), 0)
            perms = [jnp.bitwise_xor(lanes, sh) for sh in (8, 4, 2, 1)]

            def group(gi, carry2):
                i0 = gi * 16
                svec = jnp.zeros((16,), jnp.float32)
                for kk in range(16):
                    i = i0 + kk
                    acc = jnp.zeros((16,), jnp.float32)
                    for j in range(nj):
                        va = ra[i, pl.ds(j * 16, 16)]
                        vb = rb[i, pl.ds(j * 16, 16)]
                        acc = acc + jnp.maximum(va + vb, 0.0) \
                            * pw2v[pl.ds(j * 16, 16)]
                    for perm in perms:
                        acc = acc + _lane_shuffle(acc, perm)
                    svec = jnp.where(lanes == kk, acc, svec)
                sv[pl.ds(gi * 16, 16)] = svec
                return carry2

            lax.fori_loop(0, _CHP // 16, group, 0)
            pltpu.sync_copy(sv, out_h.at[pl.ds(base, _CHP)])
            return carry

        lax.fori_loop(0, nit, chunk, 0)

    return k(aarr, barr, psp, pdp, pw2flat)


def kernel(x, edge_index, pairs, W0, b0, W1, b1, W2, b2, PW1, Pb1, PW2, Pb2):
    N, din = x.shape
    hid = W0.shape[1]
    E = edge_index.shape[1]
    P = pairs.shape[0]

    npad = _ceil_to(N + 1, 2 * _RB)
    epad = _ceil_to(E, 32 * _KCH)
    ppad = _ceil_to(P, 32 * _CHP)
    half = npad // 2

    src = edge_index[0].astype(jnp.int32)
    dst = edge_index[1].astype(jnp.int32)
    ndum = npad - N
    if epad > E:
        fill = N + jnp.arange(epad - E, dtype=jnp.int32) % ndum
        src = jnp.concatenate([src, fill])
        dst = jnp.concatenate([dst, fill])
    ps = pairs[:, 0].astype(jnp.int32)
    pd = pairs[:, 1].astype(jnp.int32)
    if ppad > P:
        pfill = N + jnp.arange(ppad - P, dtype=jnp.int32) % ndum
        ps = jnp.concatenate([ps, pfill])
        pd = jnp.concatenate([pd, pfill])

    xp = jnp.pad(x, ((0, npad - N), (0, 0)))
    zacc8 = jnp.zeros(((npad + 2) * 8,), jnp.float32)
    zacc = jnp.zeros(((npad // 16 + 1) * 128,), jnp.float32)
    lut_np = np.zeros((256, 128), np.int32)
    for m in range(256):
        pos = [i for i in range(8) if (m >> i) & 1]
        lut_np[m, :len(pos)] = pos
        lut_np[m, 8] = len(pos)
    lut = jnp.asarray(lut_np)

    def scat(g2_):
        return _sc_scat(g2_, src, dst, zacc, lut, npad)\
            .reshape(2 * npad, 128)
    b0r = b0.reshape(1, hid)
    b1r = b1.reshape(1, hid)
    b2r = b2.reshape(1, hid)
    pb1r = Pb1.reshape(1, hid)
    pw1a = PW1[:hid]
    pw1b = PW1[hid:]
    pw2flat = PW2[:, 0]

    deg32r = _sc_deg(dst, zacc8, npad).reshape(32 * npad, 8)
    g2, dinv8 = _tc_first(xp, W0, deg32r, N)
    acc2 = scat(g2)
    g2 = _tc_mid(acc2, g2, dinv8, b0r, W1)
    acc2 = scat(g2)
    g2 = _tc_mid(acc2, g2, dinv8, b1r, W2)
    acc2 = scat(g2)
    aarr, barr = _tc_fin(acc2, g2, dinv8, b2r, pw1a, pw1b, pb1r)
    scores = _sc_pair(aarr, barr, ps, pd, pw2flat)
    return scores[:P] + Pb2[0]
